# baseline (speedup 1.0000x reference)
import functools

import jax
import jax.numpy as jnp
from jax import lax
from jax.experimental import pallas as pl
from jax.experimental.pallas import tpu as pltpu
from jax.experimental.pallas import tpu_sc as plsc

K = 8192
D = 256
N = 16384
BETA = 0.25
EOP_ID = 3
PAD_ID = 2

TM = 1024
TK = 512
NT = N // TM
NK = K // TK

NC = 2
NS = 16
NW = NC * NS
TOK_PER_W = N // NW
CHUNK = 128
NCHUNK = TOK_PER_W // CHUNK


GB = (0, 2736, 5472, K)


def _argmin_body(x_ref, cbT_ref, x2_ref, e2_ref, idx_ref, mind_ref,
                 gm0, gm1, gm2, ga0, ga1, ga2):
    gmin = (gm0, gm1, gm2)
    garg = (ga0, ga1, ga2)
    j = pl.program_id(1)
    x = x_ref[...]
    cbT = cbT_ref[...]
    xe = jnp.dot(x, cbT, preferred_element_type=jnp.float32)
    d = (x2_ref[...] - 2.0 * xe) + e2_ref[...]

    col = lax.broadcasted_iota(jnp.int32, (TM, TK), 1) + j * TK
    inf = jnp.float32(jnp.inf)
    for g in range(3):
        dg = jnp.where((col >= GB[g]) & (col < GB[g + 1]), d, inf)
        lmin = jnp.min(dg, axis=1, keepdims=True)
        larg = jnp.min(jnp.where(dg == lmin, col, K), axis=1,
                       keepdims=True)
        prev_min = jnp.where(j == 0, inf, gmin[g][...])
        prev_arg = jnp.where(j == 0, 0, garg[g][...])
        better = lmin < prev_min
        gmin[g][...] = jnp.where(better, lmin, prev_min)
        garg[g][...] = jnp.where(better, larg, prev_arg)

    @pl.when(j == NK - 1)
    def _():
        m0, m1, m2 = gmin[0][...], gmin[1][...], gmin[2][...]
        i0, i1, i2 = garg[0][...], garg[1][...], garg[2][...]
        r = m0.astype(jnp.bfloat16).astype(jnp.float32)
        s1 = m1 < r
        idx = jnp.where(s1, i1, i0)
        val = jnp.where(s1, m1, m0)
        carry = jnp.where(s1, m1, r)
        r2 = carry.astype(jnp.bfloat16).astype(jnp.float32)
        s2 = m2 < r2
        idx_ref[...] = jnp.where(s2, i2, idx)
        mind_ref[...] = jnp.where(s2, m2, val)


def _finalize_body(hist_ref, mind_ref, loss_ref, perp_ref):
    counts = jnp.sum(hist_ref[...], axis=0, keepdims=True)
    col = lax.broadcasted_iota(jnp.int32, (1, K), 1)
    counts = jnp.where((col == EOP_ID) | (col == PAD_ID), 0.0, counts)
    avg = counts * (1.0 / N)
    ent = jnp.sum(avg * jnp.log(avg + 1e-10))
    perp_ref[...] = jnp.exp(-ent).reshape(1, 1)
    total = jnp.sum(mind_ref[...])
    loss_ref[...] = ((1.0 + BETA) / (N * D) * total).reshape(1, 1)


def _make_sc_gather():
    mesh = plsc.VectorSubcoreMesh(core_axis_name="c", subcore_axis_name="s")

    @functools.partial(
        pl.kernel,
        mesh=mesh,
        out_type=[
            jax.ShapeDtypeStruct((N, D), jnp.float32),
            jax.ShapeDtypeStruct((NC, K), jnp.float32),
        ],
        scratch_types=[
            pltpu.VMEM((NCHUNK, CHUNK), jnp.int32),
            pltpu.VMEM((CHUNK, D), jnp.float32),
            pltpu.VMEM((CHUNK,), jnp.float32),
            pltpu.VMEM_SHARED((K,), jnp.float32),
            pltpu.SemaphoreType.DMA,
        ],
    )
    def sc_gather(idx_hbm, cb_hbm, zeros_hbm, zq_hbm, hist_hbm,
                  idx_v, rows_v, ones_v, shist, sem):
        c = lax.axis_index("c")
        s = lax.axis_index("s")
        wid = c * NS + s

        @pl.when(s == 0)
        def _():
            pltpu.sync_copy(zeros_hbm, shist)

        pltpu.sync_copy(idx_hbm.at[pl.ds(wid * NCHUNK, NCHUNK)], idx_v)
        for k in range(CHUNK // 16):
            ones_v[pl.ds(k * 16, 16)] = jnp.full((16,), 1.0, jnp.float32)

        plsc.subcore_barrier()

        base = wid * TOK_PER_W
        for j in range(NCHUNK):
            pltpu.async_copy(cb_hbm.at[idx_v.at[j]], rows_v, sem).wait()
            pltpu.sync_copy(rows_v, zq_hbm.at[pl.ds(base + j * CHUNK, CHUNK)])
            pltpu.sync_copy(ones_v, shist.at[idx_v.at[j]], add=True)

        plsc.subcore_barrier()

        @pl.when(s == 0)
        def _():
            pltpu.sync_copy(shist, hist_hbm.at[c])

    return sc_gather


_SC_CACHE = {}


def _sc_gather(idx2d, cb, zeros):
    if "fn" not in _SC_CACHE:
        _SC_CACHE["fn"] = _make_sc_gather()
    return _SC_CACHE["fn"](idx2d, cb, zeros)


def kernel(z, codebook):
    Bq, Qq, Dd = z.shape
    flat = z.reshape(N, D)
    cbT = codebook.T
    x2 = jnp.sum(flat * flat, axis=1, keepdims=True)
    e2 = jnp.sum(codebook * codebook, axis=1)[None, :]

    idx2, mind2 = pl.pallas_call(
        _argmin_body,
        grid=(NT, NK),
        in_specs=[
            pl.BlockSpec((TM, D), lambda i, j: (i, 0)),
            pl.BlockSpec((D, TK), lambda i, j: (0, j)),
            pl.BlockSpec((TM, 1), lambda i, j: (i, 0)),
            pl.BlockSpec((1, TK), lambda i, j: (0, j)),
        ],
        out_specs=[
            pl.BlockSpec((TM, 1), lambda i, j: (i, 0)),
            pl.BlockSpec((TM, 1), lambda i, j: (i, 0)),
        ],
        out_shape=[
            jax.ShapeDtypeStruct((N, 1), jnp.int32),
            jax.ShapeDtypeStruct((N, 1), jnp.float32),
        ],
        scratch_shapes=[pltpu.VMEM((TM, 1), jnp.float32)] * 3
                     + [pltpu.VMEM((TM, 1), jnp.int32)] * 3,
        compiler_params=pltpu.CompilerParams(
            dimension_semantics=("parallel", "arbitrary")),
    )(flat, cbT, x2, e2)

    indices = idx2.reshape(Bq, Qq)
    idx2d = idx2.reshape(N // CHUNK, CHUNK)
    mind3 = mind2.reshape(NT, 1, TM)
    zeros = jnp.zeros((K,), jnp.float32)

    zq_flat, hists = _sc_gather(idx2d, codebook, zeros)

    loss11, perp11 = pl.pallas_call(
        _finalize_body,
        grid=(1,),
        in_specs=[
            pl.BlockSpec((NC, K), lambda i: (0, 0)),
            pl.BlockSpec((NT, 1, TM), lambda i: (0, 0, 0)),
        ],
        out_specs=[
            pl.BlockSpec((1, 1), lambda i: (0, 0)),
            pl.BlockSpec((1, 1), lambda i: (0, 0)),
        ],
        out_shape=[
            jax.ShapeDtypeStruct((1, 1), jnp.float32),
            jax.ShapeDtypeStruct((1, 1), jnp.float32),
        ],
    )(hists, mind3)

    z_q_ste = zq_flat.reshape(Bq, Qq, Dd)
    vq_loss = loss11.reshape(())
    perplexity = perp11.reshape(())
    return (z_q_ste, vq_loss, indices, perplexity)


# speedup vs baseline: 1.7123x; 1.7123x over previous
import functools

import jax
import jax.numpy as jnp
from jax import lax
from jax.experimental import pallas as pl
from jax.experimental.pallas import tpu as pltpu
from jax.experimental.pallas import tpu_sc as plsc

K = 8192
D = 256
N = 16384
BETA = 0.25
EOP_ID = 3
PAD_ID = 2

TM = 1024
TK = 512
NT = N // TM
NK = K // TK

NC = 2
NS = 16
NW = NC * NS
TOK_PER_W = N // NW
CHUNK = 128
NCHUNK = TOK_PER_W // CHUNK


GB = (0, 2736, 5472, K)


_J_B1, _L_B1 = GB[1] // TK, GB[1] % TK
_J_B2, _L_B2 = GB[2] // TK, GB[2] % TK


def _argmin_body(x_ref, cbT_ref, x2_ref, e2_ref, idx_ref, mind_ref,
                 facc, iacc, gm0, gi0, gm1, gi1):
    j = pl.program_id(1)
    x = x_ref[...]
    cbT = cbT_ref[...]
    xe = jnp.dot(x, cbT, preferred_element_type=jnp.float32)
    d = (x2_ref[...] - 2.0 * xe) + e2_ref[...]

    inf = jnp.float32(jnp.inf)
    lane = lax.broadcasted_iota(jnp.int32, (TM, TK), 1)

    def accum(dd):
        prev = facc[...]
        better = dd < prev
        facc[...] = jnp.where(better, dd, prev)
        iacc[...] = jnp.where(better, j, iacc[...])

    def group_argmin():
        a = facc[...]
        m = jnp.min(a, axis=1, keepdims=True)
        glob = iacc[...] * TK + lane
        gi = jnp.min(jnp.where(a == m, glob, K), axis=1, keepdims=True)
        return m, gi

    @pl.when(j == 0)
    def _():
        facc[...] = d
        iacc[...] = jnp.zeros((TM, TK), jnp.int32)

    @pl.when((j != 0) & (j != _J_B1) & (j != _J_B2) & (j != NK - 1))
    def _():
        accum(d)

    @pl.when(j == _J_B1)
    def _():
        accum(jnp.where(lane < _L_B1, d, inf))
        m, gi = group_argmin()
        gm0[...] = m
        gi0[...] = gi
        facc[...] = jnp.where(lane < _L_B1, inf, d)
        iacc[...] = jnp.full((TM, TK), j, jnp.int32)

    @pl.when(j == _J_B2)
    def _():
        accum(jnp.where(lane < _L_B2, d, inf))
        m, gi = group_argmin()
        gm1[...] = m
        gi1[...] = gi
        facc[...] = jnp.where(lane < _L_B2, inf, d)
        iacc[...] = jnp.full((TM, TK), j, jnp.int32)

    @pl.when(j == NK - 1)
    def _():
        accum(d)
        m2, i2 = group_argmin()
        m0, i0 = gm0[...], gi0[...]
        m1, i1 = gm1[...], gi1[...]
        r = m0.astype(jnp.bfloat16).astype(jnp.float32)
        s1 = m1 < r
        idx = jnp.where(s1, i1, i0)
        val = jnp.where(s1, m1, m0)
        carry = jnp.where(s1, m1, r)
        r2 = carry.astype(jnp.bfloat16).astype(jnp.float32)
        s2 = m2 < r2
        idx_ref[...] = jnp.where(s2, i2, idx)
        mind_ref[...] = jnp.where(s2, m2, val)


def _finalize_body(hist_ref, mind_ref, loss_ref, perp_ref):
    counts = jnp.sum(hist_ref[...], axis=0, keepdims=True)
    col = lax.broadcasted_iota(jnp.int32, (1, K), 1)
    counts = jnp.where((col == EOP_ID) | (col == PAD_ID), 0.0, counts)
    avg = counts * (1.0 / N)
    ent = jnp.sum(avg * jnp.log(avg + 1e-10))
    perp_ref[...] = jnp.exp(-ent).reshape(1, 1)
    total = jnp.sum(mind_ref[...])
    loss_ref[...] = ((1.0 + BETA) / (N * D) * total).reshape(1, 1)


def _make_sc_gather():
    mesh = plsc.VectorSubcoreMesh(core_axis_name="c", subcore_axis_name="s")

    @functools.partial(
        pl.kernel,
        mesh=mesh,
        out_type=[
            jax.ShapeDtypeStruct((N, D), jnp.float32),
            jax.ShapeDtypeStruct((NC, K), jnp.float32),
        ],
        scratch_types=[
            pltpu.VMEM((NCHUNK, CHUNK), jnp.int32),
            pltpu.VMEM((CHUNK, D), jnp.float32),
            pltpu.VMEM((CHUNK,), jnp.float32),
            pltpu.VMEM_SHARED((K,), jnp.float32),
            pltpu.SemaphoreType.DMA,
        ],
    )
    def sc_gather(idx_hbm, cb_hbm, zeros_hbm, zq_hbm, hist_hbm,
                  idx_v, rows_v, ones_v, shist, sem):
        c = lax.axis_index("c")
        s = lax.axis_index("s")
        wid = c * NS + s

        @pl.when(s == 0)
        def _():
            pltpu.sync_copy(zeros_hbm, shist)

        pltpu.sync_copy(idx_hbm.at[pl.ds(wid * NCHUNK, NCHUNK)], idx_v)
        for k in range(CHUNK // 16):
            ones_v[pl.ds(k * 16, 16)] = jnp.full((16,), 1.0, jnp.float32)

        plsc.subcore_barrier()

        base = wid * TOK_PER_W
        for j in range(NCHUNK):
            pltpu.async_copy(cb_hbm.at[idx_v.at[j]], rows_v, sem).wait()
            pltpu.sync_copy(rows_v, zq_hbm.at[pl.ds(base + j * CHUNK, CHUNK)])
            pltpu.sync_copy(ones_v, shist.at[idx_v.at[j]], add=True)

        plsc.subcore_barrier()

        @pl.when(s == 0)
        def _():
            pltpu.sync_copy(shist, hist_hbm.at[c])

    return sc_gather


_SC_CACHE = {}


def _sc_gather(idx2d, cb, zeros):
    if "fn" not in _SC_CACHE:
        _SC_CACHE["fn"] = _make_sc_gather()
    return _SC_CACHE["fn"](idx2d, cb, zeros)


def kernel(z, codebook):
    Bq, Qq, Dd = z.shape
    flat = z.reshape(N, D)
    cbT = codebook.T
    x2 = jnp.sum(flat * flat, axis=1, keepdims=True)
    e2 = jnp.sum(codebook * codebook, axis=1)[None, :]

    idx2, mind2 = pl.pallas_call(
        _argmin_body,
        grid=(NT, NK),
        in_specs=[
            pl.BlockSpec((TM, D), lambda i, j: (i, 0)),
            pl.BlockSpec((D, TK), lambda i, j: (0, j)),
            pl.BlockSpec((TM, 1), lambda i, j: (i, 0)),
            pl.BlockSpec((1, TK), lambda i, j: (0, j)),
        ],
        out_specs=[
            pl.BlockSpec((TM, 1), lambda i, j: (i, 0)),
            pl.BlockSpec((TM, 1), lambda i, j: (i, 0)),
        ],
        out_shape=[
            jax.ShapeDtypeStruct((N, 1), jnp.int32),
            jax.ShapeDtypeStruct((N, 1), jnp.float32),
        ],
        scratch_shapes=[
            pltpu.VMEM((TM, TK), jnp.float32),
            pltpu.VMEM((TM, TK), jnp.int32),
            pltpu.VMEM((TM, 1), jnp.float32),
            pltpu.VMEM((TM, 1), jnp.int32),
            pltpu.VMEM((TM, 1), jnp.float32),
            pltpu.VMEM((TM, 1), jnp.int32),
        ],
        compiler_params=pltpu.CompilerParams(
            dimension_semantics=("parallel", "arbitrary")),
    )(flat, cbT, x2, e2)

    indices = idx2.reshape(Bq, Qq)
    idx2d = idx2.reshape(N // CHUNK, CHUNK)
    mind3 = mind2.reshape(NT, 1, TM)
    zeros = jnp.zeros((K,), jnp.float32)

    zq_flat, hists = _sc_gather(idx2d, codebook, zeros)

    loss11, perp11 = pl.pallas_call(
        _finalize_body,
        grid=(1,),
        in_specs=[
            pl.BlockSpec((NC, K), lambda i: (0, 0)),
            pl.BlockSpec((NT, 1, TM), lambda i: (0, 0, 0)),
        ],
        out_specs=[
            pl.BlockSpec((1, 1), lambda i: (0, 0)),
            pl.BlockSpec((1, 1), lambda i: (0, 0)),
        ],
        out_shape=[
            jax.ShapeDtypeStruct((1, 1), jnp.float32),
            jax.ShapeDtypeStruct((1, 1), jnp.float32),
        ],
    )(hists, mind3)

    z_q_ste = zq_flat.reshape(Bq, Qq, Dd)
    vq_loss = loss11.reshape(())
    perplexity = perp11.reshape(())
    return (z_q_ste, vq_loss, indices, perplexity)


# speedup vs baseline: 1.9191x; 1.1208x over previous
import functools

import jax
import jax.numpy as jnp
from jax import lax
from jax.experimental import pallas as pl
from jax.experimental.pallas import tpu as pltpu
from jax.experimental.pallas import tpu_sc as plsc

K = 8192
D = 256
N = 16384
BETA = 0.25
EOP_ID = 3
PAD_ID = 2

TM = 1024
TK = 1024
NT = N // TM
NK = K // TK

NC = 2
NS = 16
NW = NC * NS
TOK_PER_W = N // NW
CHUNK = 128
NCHUNK = TOK_PER_W // CHUNK


GB = (0, 2736, 5472, K)


_J_B1, _L_B1 = GB[1] // TK, GB[1] % TK
_J_B2, _L_B2 = GB[2] // TK, GB[2] % TK


def _argmin_body(x_ref, cbT_ref, x2_ref, e2_ref, idx_ref, mind_ref,
                 facc, iacc, gm0, gi0, gm1, gi1):
    j = pl.program_id(1)
    x = x_ref[...]
    cbT = cbT_ref[...]
    xe = jnp.dot(x, cbT, preferred_element_type=jnp.float32)
    d = (x2_ref[...] - 2.0 * xe) + e2_ref[...]

    inf = jnp.float32(jnp.inf)
    lane = lax.broadcasted_iota(jnp.int32, (TM, TK), 1)

    def accum(dd):
        prev = facc[...]
        better = dd < prev
        facc[...] = jnp.where(better, dd, prev)
        iacc[...] = jnp.where(better, j, iacc[...])

    def group_argmin():
        a = facc[...]
        m = jnp.min(a, axis=1, keepdims=True)
        glob = iacc[...] * TK + lane
        gi = jnp.min(jnp.where(a == m, glob, K), axis=1, keepdims=True)
        return m, gi

    @pl.when(j == 0)
    def _():
        facc[...] = d
        iacc[...] = jnp.zeros((TM, TK), jnp.int32)

    @pl.when((j != 0) & (j != _J_B1) & (j != _J_B2) & (j != NK - 1))
    def _():
        accum(d)

    @pl.when(j == _J_B1)
    def _():
        accum(jnp.where(lane < _L_B1, d, inf))
        m, gi = group_argmin()
        gm0[...] = m
        gi0[...] = gi
        facc[...] = jnp.where(lane < _L_B1, inf, d)
        iacc[...] = jnp.full((TM, TK), j, jnp.int32)

    @pl.when(j == _J_B2)
    def _():
        accum(jnp.where(lane < _L_B2, d, inf))
        m, gi = group_argmin()
        gm1[...] = m
        gi1[...] = gi
        facc[...] = jnp.where(lane < _L_B2, inf, d)
        iacc[...] = jnp.full((TM, TK), j, jnp.int32)

    @pl.when(j == NK - 1)
    def _():
        accum(d)
        m2, i2 = group_argmin()
        m0, i0 = gm0[...], gi0[...]
        m1, i1 = gm1[...], gi1[...]
        r = m0.astype(jnp.bfloat16).astype(jnp.float32)
        s1 = m1 < r
        idx = jnp.where(s1, i1, i0)
        val = jnp.where(s1, m1, m0)
        carry = jnp.where(s1, m1, r)
        r2 = carry.astype(jnp.bfloat16).astype(jnp.float32)
        s2 = m2 < r2
        idx_ref[...] = jnp.where(s2, i2, idx)
        mind_ref[...] = jnp.where(s2, m2, val)


def _finalize_body(hist_ref, mind_ref, loss_ref, perp_ref):
    counts = jnp.sum(hist_ref[...], axis=0, keepdims=True)
    col = lax.broadcasted_iota(jnp.int32, (1, K), 1)
    counts = jnp.where((col == EOP_ID) | (col == PAD_ID), 0.0, counts)
    avg = counts * (1.0 / N)
    ent = jnp.sum(avg * jnp.log(avg + 1e-10))
    perp_ref[...] = jnp.exp(-ent).reshape(1, 1)
    total = jnp.sum(mind_ref[...])
    loss_ref[...] = ((1.0 + BETA) / (N * D) * total).reshape(1, 1)


def _make_sc_gather():
    mesh = plsc.VectorSubcoreMesh(core_axis_name="c", subcore_axis_name="s")

    @functools.partial(
        pl.kernel,
        mesh=mesh,
        out_type=[
            jax.ShapeDtypeStruct((N, D), jnp.float32),
            jax.ShapeDtypeStruct((NC, K), jnp.float32),
        ],
        scratch_types=[
            pltpu.VMEM((NCHUNK, CHUNK), jnp.int32),
            pltpu.VMEM((CHUNK, D), jnp.float32),
            pltpu.VMEM((CHUNK,), jnp.float32),
            pltpu.VMEM_SHARED((K,), jnp.float32),
            pltpu.SemaphoreType.DMA,
        ],
    )
    def sc_gather(idx_hbm, cb_hbm, zeros_hbm, zq_hbm, hist_hbm,
                  idx_v, rows_v, ones_v, shist, sem):
        c = lax.axis_index("c")
        s = lax.axis_index("s")
        wid = c * NS + s

        @pl.when(s == 0)
        def _():
            pltpu.sync_copy(zeros_hbm, shist)

        pltpu.sync_copy(idx_hbm.at[pl.ds(wid * NCHUNK, NCHUNK)], idx_v)
        for k in range(CHUNK // 16):
            ones_v[pl.ds(k * 16, 16)] = jnp.full((16,), 1.0, jnp.float32)

        plsc.subcore_barrier()

        base = wid * TOK_PER_W
        for j in range(NCHUNK):
            pltpu.async_copy(cb_hbm.at[idx_v.at[j]], rows_v, sem).wait()
            pltpu.sync_copy(rows_v, zq_hbm.at[pl.ds(base + j * CHUNK, CHUNK)])
            pltpu.sync_copy(ones_v, shist.at[idx_v.at[j]], add=True)

        plsc.subcore_barrier()

        @pl.when(s == 0)
        def _():
            pltpu.sync_copy(shist, hist_hbm.at[c])

    return sc_gather


_SC_CACHE = {}


def _sc_gather(idx2d, cb, zeros):
    if "fn" not in _SC_CACHE:
        _SC_CACHE["fn"] = _make_sc_gather()
    return _SC_CACHE["fn"](idx2d, cb, zeros)


def kernel(z, codebook):
    Bq, Qq, Dd = z.shape
    flat = z.reshape(N, D)
    cbT = codebook.T
    x2 = jnp.sum(flat * flat, axis=1, keepdims=True)
    e2 = jnp.sum(codebook * codebook, axis=1)[None, :]

    idx2, mind2 = pl.pallas_call(
        _argmin_body,
        grid=(NT, NK),
        in_specs=[
            pl.BlockSpec((TM, D), lambda i, j: (i, 0)),
            pl.BlockSpec((D, TK), lambda i, j: (0, j)),
            pl.BlockSpec((TM, 1), lambda i, j: (i, 0)),
            pl.BlockSpec((1, TK), lambda i, j: (0, j)),
        ],
        out_specs=[
            pl.BlockSpec((TM, 1), lambda i, j: (i, 0)),
            pl.BlockSpec((TM, 1), lambda i, j: (i, 0)),
        ],
        out_shape=[
            jax.ShapeDtypeStruct((N, 1), jnp.int32),
            jax.ShapeDtypeStruct((N, 1), jnp.float32),
        ],
        scratch_shapes=[
            pltpu.VMEM((TM, TK), jnp.float32),
            pltpu.VMEM((TM, TK), jnp.int32),
            pltpu.VMEM((TM, 1), jnp.float32),
            pltpu.VMEM((TM, 1), jnp.int32),
            pltpu.VMEM((TM, 1), jnp.float32),
            pltpu.VMEM((TM, 1), jnp.int32),
        ],
        compiler_params=pltpu.CompilerParams(
            dimension_semantics=("parallel", "arbitrary")),
    )(flat, cbT, x2, e2)

    indices = idx2.reshape(Bq, Qq)
    idx2d = idx2.reshape(N // CHUNK, CHUNK)
    mind3 = mind2.reshape(NT, 1, TM)
    zeros = jnp.zeros((K,), jnp.float32)

    zq_flat, hists = _sc_gather(idx2d, codebook, zeros)

    loss11, perp11 = pl.pallas_call(
        _finalize_body,
        grid=(1,),
        in_specs=[
            pl.BlockSpec((NC, K), lambda i: (0, 0)),
            pl.BlockSpec((NT, 1, TM), lambda i: (0, 0, 0)),
        ],
        out_specs=[
            pl.BlockSpec((1, 1), lambda i: (0, 0)),
            pl.BlockSpec((1, 1), lambda i: (0, 0)),
        ],
        out_shape=[
            jax.ShapeDtypeStruct((1, 1), jnp.float32),
            jax.ShapeDtypeStruct((1, 1), jnp.float32),
        ],
    )(hists, mind3)

    z_q_ste = zq_flat.reshape(Bq, Qq, Dd)
    vq_loss = loss11.reshape(())
    perplexity = perp11.reshape(())
    return (z_q_ste, vq_loss, indices, perplexity)


# speedup vs baseline: 2.0568x; 1.0718x over previous
import functools

import jax
import jax.numpy as jnp
from jax import lax
from jax.experimental import pallas as pl
from jax.experimental.pallas import tpu as pltpu
from jax.experimental.pallas import tpu_sc as plsc

K = 8192
D = 256
N = 16384
BETA = 0.25
EOP_ID = 3
PAD_ID = 2

TM = 2048
TK = 1024
NT = N // TM
NK = K // TK

NC = 2
NS = 16
NW = NC * NS
TOK_PER_W = N // NW
CHUNK = 128
NCHUNK = TOK_PER_W // CHUNK


GB = (0, 2736, 5472, K)


_J_B1, _L_B1 = GB[1] // TK, GB[1] % TK
_J_B2, _L_B2 = GB[2] // TK, GB[2] % TK


def _argmin_body(x_ref, cbT_ref, x2_ref, e2_ref, idx_ref, mind_ref,
                 facc, iacc, gm0, gi0, gm1, gi1):
    j = pl.program_id(1)
    x = x_ref[...]
    cbT = cbT_ref[...]
    xe = jnp.dot(x, cbT, preferred_element_type=jnp.float32)
    d = (x2_ref[...] - 2.0 * xe) + e2_ref[...]

    inf = jnp.float32(jnp.inf)
    lane = lax.broadcasted_iota(jnp.int32, (TM, TK), 1)

    def accum(dd):
        prev = facc[...]
        better = dd < prev
        facc[...] = jnp.where(better, dd, prev)
        iacc[...] = jnp.where(better, j, iacc[...])

    def group_argmin():
        a = facc[...]
        m = jnp.min(a, axis=1, keepdims=True)
        glob = iacc[...] * TK + lane
        gi = jnp.min(jnp.where(a == m, glob, K), axis=1, keepdims=True)
        return m, gi

    @pl.when(j == 0)
    def _():
        facc[...] = d
        iacc[...] = jnp.zeros((TM, TK), jnp.int32)

    @pl.when((j != 0) & (j != _J_B1) & (j != _J_B2) & (j != NK - 1))
    def _():
        accum(d)

    @pl.when(j == _J_B1)
    def _():
        accum(jnp.where(lane < _L_B1, d, inf))
        m, gi = group_argmin()
        gm0[...] = m
        gi0[...] = gi
        facc[...] = jnp.where(lane < _L_B1, inf, d)
        iacc[...] = jnp.full((TM, TK), j, jnp.int32)

    @pl.when(j == _J_B2)
    def _():
        accum(jnp.where(lane < _L_B2, d, inf))
        m, gi = group_argmin()
        gm1[...] = m
        gi1[...] = gi
        facc[...] = jnp.where(lane < _L_B2, inf, d)
        iacc[...] = jnp.full((TM, TK), j, jnp.int32)

    @pl.when(j == NK - 1)
    def _():
        accum(d)
        m2, i2 = group_argmin()
        m0, i0 = gm0[...], gi0[...]
        m1, i1 = gm1[...], gi1[...]
        r = m0.astype(jnp.bfloat16).astype(jnp.float32)
        s1 = m1 < r
        idx = jnp.where(s1, i1, i0)
        val = jnp.where(s1, m1, m0)
        carry = jnp.where(s1, m1, r)
        r2 = carry.astype(jnp.bfloat16).astype(jnp.float32)
        s2 = m2 < r2
        idx_ref[...] = jnp.where(s2, i2, idx)
        mind_ref[...] = jnp.where(s2, m2, val)


def _finalize_body(hist_ref, mind_ref, loss_ref, perp_ref):
    counts = jnp.sum(hist_ref[...], axis=0, keepdims=True)
    col = lax.broadcasted_iota(jnp.int32, (1, K), 1)
    counts = jnp.where((col == EOP_ID) | (col == PAD_ID), 0.0, counts)
    avg = counts * (1.0 / N)
    ent = jnp.sum(avg * jnp.log(avg + 1e-10))
    perp_ref[...] = jnp.exp(-ent).reshape(1, 1)
    total = jnp.sum(mind_ref[...])
    loss_ref[...] = ((1.0 + BETA) / (N * D) * total).reshape(1, 1)


def _make_sc_gather(ntok):
    tok_per_w = ntok // NW
    nchunk = tok_per_w // CHUNK
    mesh = plsc.VectorSubcoreMesh(core_axis_name="c", subcore_axis_name="s")

    @functools.partial(
        pl.kernel,
        mesh=mesh,
        out_type=[
            jax.ShapeDtypeStruct((ntok, D), jnp.float32),
            jax.ShapeDtypeStruct((NC, K), jnp.float32),
        ],
        scratch_types=[
            pltpu.VMEM((nchunk, CHUNK), jnp.int32),
            pltpu.VMEM((CHUNK, D), jnp.float32),
            pltpu.VMEM((CHUNK,), jnp.float32),
            pltpu.VMEM_SHARED((K,), jnp.float32),
            pltpu.SemaphoreType.DMA,
        ],
    )
    def sc_gather(idx_hbm, cb_hbm, zeros_hbm, zq_hbm, hist_hbm,
                  idx_v, rows_v, ones_v, shist, sem):
        c = lax.axis_index("c")
        s = lax.axis_index("s")
        wid = c * NS + s

        @pl.when(s == 0)
        def _():
            pltpu.sync_copy(zeros_hbm, shist)

        pltpu.sync_copy(idx_hbm.at[pl.ds(wid * nchunk, nchunk)], idx_v)
        for k in range(CHUNK // 16):
            ones_v[pl.ds(k * 16, 16)] = jnp.full((16,), 1.0, jnp.float32)

        plsc.subcore_barrier()

        base = wid * tok_per_w
        for j in range(nchunk):
            pltpu.async_copy(cb_hbm.at[idx_v.at[j]], rows_v, sem).wait()
            pltpu.sync_copy(rows_v, zq_hbm.at[pl.ds(base + j * CHUNK, CHUNK)])
            pltpu.sync_copy(ones_v, shist.at[idx_v.at[j]], add=True)

        plsc.subcore_barrier()

        @pl.when(s == 0)
        def _():
            pltpu.sync_copy(shist, hist_hbm.at[c])

    return sc_gather


_SC_CACHE = {}


def _sc_gather(idx2d, cb, zeros):
    ntok = idx2d.shape[0] * idx2d.shape[1]
    if ntok not in _SC_CACHE:
        _SC_CACHE[ntok] = _make_sc_gather(ntok)
    return _SC_CACHE[ntok](idx2d, cb, zeros)


def _argmin_call(flat_h, cbT, x2_h, e2):
    return pl.pallas_call(
        _argmin_body,
        grid=(NT, NK),
        in_specs=[
            pl.BlockSpec((TM, D), lambda i, j: (i, 0)),
            pl.BlockSpec((D, TK), lambda i, j: (0, j)),
            pl.BlockSpec((TM, 1), lambda i, j: (i, 0)),
            pl.BlockSpec((1, TK), lambda i, j: (0, j)),
        ],
        out_specs=[
            pl.BlockSpec((TM, 1), lambda i, j: (i, 0)),
            pl.BlockSpec((TM, 1), lambda i, j: (i, 0)),
        ],
        out_shape=[
            jax.ShapeDtypeStruct((N, 1), jnp.int32),
            jax.ShapeDtypeStruct((N, 1), jnp.float32),
        ],
        scratch_shapes=[
            pltpu.VMEM((TM, TK), jnp.float32),
            pltpu.VMEM((TM, TK), jnp.int32),
            pltpu.VMEM((TM, 1), jnp.float32),
            pltpu.VMEM((TM, 1), jnp.int32),
            pltpu.VMEM((TM, 1), jnp.float32),
            pltpu.VMEM((TM, 1), jnp.int32),
        ],
        compiler_params=pltpu.CompilerParams(
            dimension_semantics=("parallel", "arbitrary")),
    )(flat_h, cbT, x2_h, e2)


def kernel(z, codebook):
    Bq, Qq, Dd = z.shape
    flat = z.reshape(N, D)
    cbT = codebook.T
    x2 = jnp.sum(flat * flat, axis=1, keepdims=True)
    e2 = jnp.sum(codebook * codebook, axis=1)[None, :]
    zeros = jnp.zeros((K,), jnp.float32)

    idx2, mind2 = _argmin_call(flat, cbT, x2, e2)

    zq_flat, hists = _sc_gather(idx2.reshape(N // CHUNK, CHUNK),
                                codebook, zeros)

    indices = idx2.reshape(Bq, Qq)
    mind3 = mind2.reshape(NT, 1, TM)

    loss11, perp11 = pl.pallas_call(
        _finalize_body,
        grid=(1,),
        in_specs=[
            pl.BlockSpec((NC, K), lambda i: (0, 0)),
            pl.BlockSpec((NT, 1, TM), lambda i: (0, 0, 0)),
        ],
        out_specs=[
            pl.BlockSpec((1, 1), lambda i: (0, 0)),
            pl.BlockSpec((1, 1), lambda i: (0, 0)),
        ],
        out_shape=[
            jax.ShapeDtypeStruct((1, 1), jnp.float32),
            jax.ShapeDtypeStruct((1, 1), jnp.float32),
        ],
    )(hists, mind3)

    z_q_ste = zq_flat.reshape(Bq, Qq, Dd)
    vq_loss = loss11.reshape(())
    perplexity = perp11.reshape(())
    return (z_q_ste, vq_loss, indices, perplexity)
